# Initial kernel scaffold; baseline (speedup 1.0000x reference)
import jax
import jax.numpy as jnp
from jax.experimental import pallas as pl


def kernel(node_features, pos, fc_w1, fc_w2):
    raise NotImplementedError("write your pallas kernel here")


# speedup vs baseline: 3.9539x; 3.9539x over previous
import numpy as np
import jax
import jax.numpy as jnp
from jax import lax
from jax.experimental import pallas as pl
from jax.experimental.pallas import tpu as pltpu

_R = 0.15
_NB = 10
_COEF = 1.14136 * float(np.exp(2.0))
_STEP = _R / (_NB + 1)


def _sus(x):
    safe = jnp.where(x > 0.0, x, 1.0)
    return jnp.where(x > 0.0, jnp.exp(-1.0 / safe), 0.0)


def _body(posT_ref, post_ref, x_ref, w1_ref, wc_ref, out_ref, d0, d1, d2):
    tt = pl.program_id(1)

    @pl.when(tt == 0)
    def _():
        x = x_ref[0]
        d0[...] = jnp.dot(x, wc_ref[0], preferred_element_type=jnp.float32)
        d1[...] = jnp.dot(x, wc_ref[1], preferred_element_type=jnp.float32)
        d2[...] = jnp.dot(x, wc_ref[2], preferred_element_type=jnp.float32)

    pt = post_ref[0]
    ptx = pt[:, 0:1]
    pty = pt[:, 1:2]
    ptz = pt[:, 2:3]
    trow = tt * 128 + lax.broadcasted_iota(jnp.int32, (128, 128), 0)

    def sbody(sc, acc):
        base = sc * 128
        dx = ptx - posT_ref[0, 0:1, pl.ds(base, 128)]
        dy = pty - posT_ref[0, 1:2, pl.ds(base, 128)]
        dz = ptz - posT_ref[0, 2:3, pl.ds(base, 128)]
        d = jnp.sqrt(dx * dx + dy * dy + dz * dz)
        scol = base + lax.broadcasted_iota(jnp.int32, (128, 128), 1)
        gate = jnp.where((d < _R) & (trow != scol),
                         _sus(10.0 * (1.0 - d / _R)), 0.0)
        parts = []
        for b in range(_NB):
            u = (d - (b + 1) * _STEP) / _STEP
            parts.append(_COEF * _sus(u + 1.0) * _sus(1.0 - u))
        parts.append(gate)
        eg = jnp.stack(parts, axis=-1).reshape(16384, _NB + 1)
        a = jnp.dot(eg[:, :_NB], w1_ref[...],
                    preferred_element_type=jnp.float32)
        gh = a * jax.nn.sigmoid(a) * eg[:, _NB:_NB + 1]
        gh3 = gh.reshape(128, 128, 64)
        accs = []
        for dsc in (d0, d1, d2):
            prod = gh3 * dsc[pl.ds(base, 128), :][None]
            accs.append(jnp.sum(jnp.sum(prod, axis=2), axis=1, keepdims=True))
        return acc + jnp.concatenate(accs, axis=1)

    acc = lax.fori_loop(0, 8, sbody, jnp.zeros((128, 3), jnp.float32))
    out_ref[0] = acc * (1.0 / 32.0)


def kernel(node_features, pos, fc_w1, fc_w2):
    B, N, F = node_features.shape
    w = fc_w2[:, :512].reshape(64, 16, 32)[:, :, :3]
    wc = jnp.stack([w[..., 2] - w[..., 1],
                    w[..., 0] - w[..., 2],
                    w[..., 1] - w[..., 0]], axis=-1)
    wcp = wc.transpose(2, 1, 0)
    posT = pos.transpose(0, 2, 1)

    grid = (B, N // 128)
    out = pl.pallas_call(
        _body,
        grid=grid,
        in_specs=[
            pl.BlockSpec((1, 3, N), lambda b, t: (b, 0, 0)),
            pl.BlockSpec((1, 128, 3), lambda b, t: (b, t, 0)),
            pl.BlockSpec((1, N, F), lambda b, t: (b, 0, 0)),
            pl.BlockSpec((_NB, 64), lambda b, t: (0, 0)),
            pl.BlockSpec((3, F, 64), lambda b, t: (0, 0, 0)),
        ],
        out_specs=pl.BlockSpec((1, 128, 3), lambda b, t: (b, t, 0)),
        out_shape=jax.ShapeDtypeStruct((B, N, 3), jnp.float32),
        scratch_shapes=[pltpu.VMEM((N, 64), jnp.float32) for _ in range(3)],
    )(posT, pos, node_features, fc_w1, wcp)
    return out
